# initial kernel scaffold (unmeasured)
import jax
import jax.numpy as jnp
from jax import lax
from jax.experimental import pallas as pl
from jax.experimental.pallas import tpu as pltpu

B, SQ, H, D = 4, 256, 16, 64
SCALE = D ** -0.5


def _comm_body(kv_ref, out_ref, send_sem, recv_sem):
    x = lax.axis_index("x")
    y = lax.axis_index("y")
    z = lax.axis_index("z")
    nbr = (x, 1 - y, z)

    bar = pltpu.get_barrier_semaphore()
    pl.semaphore_signal(bar, inc=1, device_id=nbr,
                        device_id_type=pl.DeviceIdType.MESH)
    pl.semaphore_wait(bar, 1)

    rdma = pltpu.make_async_remote_copy(
        src_ref=kv_ref,
        dst_ref=out_ref,
        send_sem=send_sem,
        recv_sem=recv_sem,
        device_id=nbr,
        device_id_type=pl.DeviceIdType.MESH,
    )
    rdma.start()
    rdma.wait()


def _attn_body(q_ref, k_ref, v_ref, kv_ref, out_ref):
    q = q_ref[0, :, 0, :]
    k = jnp.concatenate([k_ref[0, :, 0, :], kv_ref[0, 0, :, 0, :]], axis=0)
    v = jnp.concatenate([v_ref[0, :, 0, :], kv_ref[1, 0, :, 0, :]], axis=0)
    s = lax.dot_general(q, k, (((1,), (1,)), ((), ())),
                        preferred_element_type=jnp.float32) * SCALE
    m = jnp.max(s, axis=-1, keepdims=True)
    p = jnp.exp(s - m)
    p = p / jnp.sum(p, axis=-1, keepdims=True)
    o = lax.dot_general(p.astype(jnp.bfloat16), v, (((1,), (0,)), ((), ())),
                        preferred_element_type=jnp.float32)
    out_ref[0, :, 0, :] = o


def kernel(Q, K, V):
    Qb = Q.astype(jnp.bfloat16)
    Kb = K.astype(jnp.bfloat16)
    Vb = V.astype(jnp.bfloat16)
    kv_loc = jnp.stack([Kb, Vb])

    kv_rem = pl.pallas_call(
        _comm_body,
        out_shape=jax.ShapeDtypeStruct((2, B, SQ, H, D), jnp.bfloat16),
        in_specs=[pl.BlockSpec(memory_space=pltpu.VMEM)],
        out_specs=pl.BlockSpec(memory_space=pltpu.VMEM),
        scratch_shapes=[pltpu.SemaphoreType.DMA, pltpu.SemaphoreType.DMA],
        compiler_params=pltpu.CompilerParams(collective_id=0),
    )(kv_loc)

    out = pl.pallas_call(
        _attn_body,
        grid=(B, H),
        in_specs=[
            pl.BlockSpec((1, SQ, 1, D), lambda b, h: (b, 0, h, 0)),
            pl.BlockSpec((1, SQ, 1, D), lambda b, h: (b, 0, h, 0)),
            pl.BlockSpec((1, SQ, 1, D), lambda b, h: (b, 0, h, 0)),
            pl.BlockSpec((2, 1, SQ, 1, D), lambda b, h: (0, b, 0, h, 0)),
        ],
        out_specs=pl.BlockSpec((1, SQ, 1, D), lambda b, h: (b, 0, h, 0)),
        out_shape=jax.ShapeDtypeStruct((B, SQ, H, D), jnp.float32),
    )(Qb, Kb, Vb, kv_rem)
    return out


# baseline (device time: 163515 ns/iter reference)
import jax
import jax.numpy as jnp
from jax import lax
from jax.experimental import pallas as pl
from jax.experimental.pallas import tpu as pltpu

B, SQ, H, D = 4, 256, 16, 64
SCALE = D ** -0.5


def _comm_body(kv_ref, out_ref, send_sem, recv_sem):
    x = lax.axis_index("x")
    y = lax.axis_index("y")
    z = lax.axis_index("z")
    nbr = (x, 1 - y, z)

    bar = pltpu.get_barrier_semaphore()
    pl.semaphore_signal(bar, inc=1, device_id=nbr,
                        device_id_type=pl.DeviceIdType.MESH)
    pl.semaphore_wait(bar, 1)

    rdma = pltpu.make_async_remote_copy(
        src_ref=kv_ref,
        dst_ref=out_ref,
        send_sem=send_sem,
        recv_sem=recv_sem,
        device_id=nbr,
        device_id_type=pl.DeviceIdType.MESH,
    )
    rdma.start()
    rdma.wait()


def _attn_body(q_ref, k_ref, v_ref, kv_ref, out_ref):
    q = q_ref[0, 0]
    k = jnp.concatenate([k_ref[0, 0], kv_ref[0, 0, 0]], axis=0)
    v = jnp.concatenate([v_ref[0, 0], kv_ref[1, 0, 0]], axis=0)
    s = lax.dot_general(q, k, (((1,), (1,)), ((), ())),
                        preferred_element_type=jnp.float32) * SCALE
    m = jnp.max(s, axis=-1, keepdims=True)
    p = jnp.exp(s - m)
    p = p / jnp.sum(p, axis=-1, keepdims=True)
    o = lax.dot_general(p.astype(jnp.bfloat16), v, (((1,), (0,)), ((), ())),
                        preferred_element_type=jnp.float32)
    out_ref[0, 0] = o


def kernel(Q, K, V):
    Qt = Q.astype(jnp.bfloat16).transpose(0, 2, 1, 3)
    Kt = K.astype(jnp.bfloat16).transpose(0, 2, 1, 3)
    Vt = V.astype(jnp.bfloat16).transpose(0, 2, 1, 3)
    kv_loc = jnp.stack([Kt, Vt])

    kv_rem = pl.pallas_call(
        _comm_body,
        out_shape=jax.ShapeDtypeStruct((2, B, H, SQ, D), jnp.bfloat16),
        in_specs=[pl.BlockSpec(memory_space=pltpu.VMEM)],
        out_specs=pl.BlockSpec(memory_space=pltpu.VMEM),
        scratch_shapes=[pltpu.SemaphoreType.DMA, pltpu.SemaphoreType.DMA],
        compiler_params=pltpu.CompilerParams(collective_id=0),
    )(kv_loc)

    out = pl.pallas_call(
        _attn_body,
        grid=(B, H),
        in_specs=[
            pl.BlockSpec((1, 1, SQ, D), lambda b, h: (b, h, 0, 0)),
            pl.BlockSpec((1, 1, SQ, D), lambda b, h: (b, h, 0, 0)),
            pl.BlockSpec((1, 1, SQ, D), lambda b, h: (b, h, 0, 0)),
            pl.BlockSpec((2, 1, 1, SQ, D), lambda b, h: (0, b, h, 0, 0)),
        ],
        out_specs=pl.BlockSpec((1, 1, SQ, D), lambda b, h: (b, h, 0, 0)),
        out_shape=jax.ShapeDtypeStruct((B, H, SQ, D), jnp.float32),
    )(Qt, Kt, Vt, kv_rem)
    return out.transpose(0, 2, 1, 3)


# device time: 93647 ns/iter; 1.7461x vs baseline; 1.7461x over previous
import jax
import jax.numpy as jnp
from jax import lax
from jax.experimental import pallas as pl
from jax.experimental.pallas import tpu as pltpu

B, SQ, H, D = 4, 256, 16, 64
SCALE = D ** -0.5
KV_ROWS = 2 * B * H * SQ * D // 512


def _comm_body(kv_ref, out_ref, send_sem, recv_sem):
    x = lax.axis_index("x")
    y = lax.axis_index("y")
    z = lax.axis_index("z")
    nbr = (x, 1 - y, z)

    bar = pltpu.get_barrier_semaphore()
    pl.semaphore_signal(bar, inc=1, device_id=nbr,
                        device_id_type=pl.DeviceIdType.MESH)
    pl.semaphore_wait(bar, 1)

    rdma = pltpu.make_async_remote_copy(
        src_ref=kv_ref,
        dst_ref=out_ref,
        send_sem=send_sem,
        recv_sem=recv_sem,
        device_id=nbr,
        device_id_type=pl.DeviceIdType.MESH,
    )
    rdma.start()
    rdma.wait()


def _attn_body(q_ref, k_ref, v_ref, kr_ref, vr_ref, out_ref):
    q = q_ref[0, 0] * SCALE

    def part(k, v):
        s_t = lax.dot_general(k, q, (((0,), (0,)), ((), ())),
                              preferred_element_type=jnp.float32)
        p = jnp.exp(s_t)
        l = jnp.sum(p, axis=0, keepdims=True)
        o = lax.dot_general(v, p.astype(jnp.bfloat16),
                            (((1,), (0,)), ((), ())),
                            preferred_element_type=jnp.float32)
        return o, l

    o1, l1 = part(k_ref[0, 0], v_ref[0, 0])
    o2, l2 = part(kr_ref[0, 0], vr_ref[0, 0])
    out_ref[0, 0] = (o1 + o2) * (1.0 / (l1 + l2))


def kernel(Q, K, V):
    Qt = Q.astype(jnp.bfloat16).transpose(0, 2, 3, 1)
    Kt = K.astype(jnp.bfloat16).transpose(0, 2, 3, 1)
    Vt = V.astype(jnp.bfloat16).transpose(0, 2, 3, 1)
    kv_flat = jnp.concatenate(
        [Kt.reshape(-1, 512), Vt.reshape(-1, 512)], axis=0)

    kv_rem = pl.pallas_call(
        _comm_body,
        out_shape=jax.ShapeDtypeStruct((KV_ROWS, 512), jnp.bfloat16),
        in_specs=[pl.BlockSpec(memory_space=pltpu.VMEM)],
        out_specs=pl.BlockSpec(memory_space=pltpu.VMEM),
        scratch_shapes=[pltpu.SemaphoreType.DMA, pltpu.SemaphoreType.DMA],
        compiler_params=pltpu.CompilerParams(collective_id=0),
    )(kv_flat)

    krem = kv_rem[: KV_ROWS // 2].reshape(B, H, D, SQ)
    vrem = kv_rem[KV_ROWS // 2:].reshape(B, H, D, SQ)

    spec = pl.BlockSpec((1, 1, D, SQ), lambda b, h: (b, h, 0, 0))
    out = pl.pallas_call(
        _attn_body,
        grid=(B, H),
        in_specs=[spec, spec, spec, spec, spec],
        out_specs=spec,
        out_shape=jax.ShapeDtypeStruct((B, H, D, SQ), jnp.float32),
    )(Qt, Kt, Vt, krem, vrem)
    return out.transpose(0, 3, 1, 2)


# device time: 76638 ns/iter; 2.1336x vs baseline; 1.2219x over previous
import jax
import jax.numpy as jnp
from jax import lax
from jax.experimental import pallas as pl
from jax.experimental.pallas import tpu as pltpu

B, SQ, H, D = 4, 256, 16, 64
SCALE = D ** -0.5
KV_ROWS = 2 * B * H * SQ * D // 512


C = 8
HALF = KV_ROWS // 2
R = HALF // C


def _comm_body(kv_ref, out_ref, ys_sem, yr_sem, xs_sem, xr_sem):
    x = lax.axis_index("x")
    y = lax.axis_index("y")
    z = lax.axis_index("z")
    ynbr = (x, 1 - y, z)
    xnbr = (1 - x, y, z)
    base = x * HALF

    bar = pltpu.get_barrier_semaphore()
    pl.semaphore_signal(bar, inc=1, device_id=ynbr,
                        device_id_type=pl.DeviceIdType.MESH)
    pl.semaphore_signal(bar, inc=1, device_id=xnbr,
                        device_id_type=pl.DeviceIdType.MESH)
    pl.semaphore_wait(bar, 2)

    y_rdmas = []
    for i in range(C):
        sl = pl.ds(base + i * R, R)
        r = pltpu.make_async_remote_copy(
            src_ref=kv_ref.at[sl], dst_ref=out_ref.at[sl],
            send_sem=ys_sem.at[i], recv_sem=yr_sem.at[i],
            device_id=ynbr, device_id_type=pl.DeviceIdType.MESH)
        r.start()
        y_rdmas.append(r)

    x_rdmas = []
    for i in range(C):
        y_rdmas[i].wait_recv()
        sl = pl.ds(base + i * R, R)
        r = pltpu.make_async_remote_copy(
            src_ref=out_ref.at[sl], dst_ref=out_ref.at[sl],
            send_sem=xs_sem.at[i], recv_sem=xr_sem.at[i],
            device_id=xnbr, device_id_type=pl.DeviceIdType.MESH)
        r.start()
        x_rdmas.append(r)

    for r in x_rdmas:
        r.wait_recv()
    for i in range(C):
        y_rdmas[i].wait_send()
        x_rdmas[i].wait_send()


def _attn_body(q_ref, k_ref, v_ref, kr_ref, vr_ref, out_ref):
    q = q_ref[0, 0] * SCALE

    def part(k, v):
        s_t = lax.dot_general(k, q, (((0,), (0,)), ((), ())),
                              preferred_element_type=jnp.float32)
        p = jnp.exp(s_t)
        l = jnp.sum(p, axis=0, keepdims=True)
        o = lax.dot_general(v, p.astype(jnp.bfloat16),
                            (((1,), (0,)), ((), ())),
                            preferred_element_type=jnp.float32)
        return o, l

    o1, l1 = part(k_ref[0, 0], v_ref[0, 0])
    o2, l2 = part(kr_ref[0, 0], vr_ref[0, 0])
    out_ref[0, 0] = (o1 + o2) * (1.0 / (l1 + l2))


def kernel(Q, K, V):
    Qt = Q.astype(jnp.bfloat16).transpose(0, 2, 3, 1)
    Kt = K.astype(jnp.bfloat16).transpose(0, 2, 3, 1)
    Vt = V.astype(jnp.bfloat16).transpose(0, 2, 3, 1)
    kv_flat = jnp.concatenate(
        [Kt.reshape(-1, 512), Vt.reshape(-1, 512)], axis=0)

    kv_rem = pl.pallas_call(
        _comm_body,
        out_shape=jax.ShapeDtypeStruct((KV_ROWS, 512), jnp.bfloat16),
        in_specs=[pl.BlockSpec(memory_space=pltpu.VMEM)],
        out_specs=pl.BlockSpec(memory_space=pltpu.VMEM),
        scratch_shapes=[pltpu.SemaphoreType.DMA((C,)),
                        pltpu.SemaphoreType.DMA((C,)),
                        pltpu.SemaphoreType.DMA((C,)),
                        pltpu.SemaphoreType.DMA((C,))],
        compiler_params=pltpu.CompilerParams(collective_id=0),
    )(kv_flat)

    krem = kv_rem[: KV_ROWS // 2].reshape(B, H, D, SQ)
    vrem = kv_rem[KV_ROWS // 2:].reshape(B, H, D, SQ)

    spec = pl.BlockSpec((1, 1, D, SQ), lambda b, h: (b, h, 0, 0))
    out = pl.pallas_call(
        _attn_body,
        grid=(B, H),
        in_specs=[spec, spec, spec, spec, spec],
        out_specs=spec,
        out_shape=jax.ShapeDtypeStruct((B, H, D, SQ), jnp.float32),
    )(Qt, Kt, Vt, krem, vrem)
    return out.transpose(0, 3, 1, 2)


# device time: 61585 ns/iter; 2.6551x vs baseline; 1.2444x over previous
import jax
import jax.numpy as jnp
from jax import lax
from jax.experimental import pallas as pl
from jax.experimental.pallas import tpu as pltpu

B, SQ, H, D = 4, 256, 16, 64
HH = H // 2
SCALE = D ** -0.5
NCHUNK = 2 * B


def _part(q, k, v):
    s_t = lax.dot_general(k, q, (((0,), (0,)), ((), ())),
                          preferred_element_type=jnp.float32)
    p = jnp.exp(s_t)
    l = jnp.sum(p, axis=0, keepdims=True)
    o = lax.dot_general(v, p.astype(jnp.bfloat16), (((1,), (0,)), ((), ())),
                        preferred_element_type=jnp.float32)
    return o, l


def _body(q_ref, kv_ref, out_ref, kvr_ref, l_ref,
          ys_sem, yr_sem, xs_sem, xr_sem):
    x = lax.axis_index("x")
    y = lax.axis_index("y")
    z = lax.axis_index("z")
    ynbr = (x, 1 - y, z)
    xnbr = (1 - x, y, z)
    my_h = pl.ds(x * HH, HH)
    ot_h = pl.ds((1 - x) * HH, HH)

    bar = pltpu.get_barrier_semaphore()
    pl.semaphore_signal(bar, inc=1, device_id=ynbr,
                        device_id_type=pl.DeviceIdType.MESH)
    pl.semaphore_signal(bar, inc=1, device_id=xnbr,
                        device_id_type=pl.DeviceIdType.MESH)
    pl.semaphore_wait(bar, 2)

    y_rdmas = []
    for b in range(B):
        for kvi in range(2):
            i = 2 * b + kvi
            r = pltpu.make_async_remote_copy(
                src_ref=kv_ref.at[kvi, b, my_h],
                dst_ref=kvr_ref.at[kvi, b, my_h],
                send_sem=ys_sem.at[i], recv_sem=yr_sem.at[i],
                device_id=ynbr, device_id_type=pl.DeviceIdType.MESH)
            r.start()
            y_rdmas.append(r)

    x_rdmas = []
    for b in range(B):
        for h in range(H):
            q = q_ref[b, h] * SCALE
            o1, l1 = _part(q, kv_ref[0, b, h], kv_ref[1, b, h])
            out_ref[b, h] = o1
            l_ref[pl.ds(b * H + h, 1)] = l1
        for kvi in range(2):
            i = 2 * b + kvi
            y_rdmas[i].wait_recv()
            r = pltpu.make_async_remote_copy(
                src_ref=kvr_ref.at[kvi, b, my_h],
                dst_ref=kvr_ref.at[kvi, b, my_h],
                send_sem=xs_sem.at[i], recv_sem=xr_sem.at[i],
                device_id=xnbr, device_id_type=pl.DeviceIdType.MESH)
            r.start()
            x_rdmas.append(r)

    def _fold_remote(b, h):
        q = q_ref[b, h] * SCALE
        o2, l2 = _part(q, kvr_ref[0, b, h], kvr_ref[1, b, h])
        bh = pl.ds(b * H + h, 1)
        out_ref[b, h] = (out_ref[b, h] + o2) * (1.0 / (l_ref[bh] + l2))

    for b in range(B):
        for hh in range(HH):
            @pl.when(x == 0)
            def _():
                _fold_remote(b, hh)

            @pl.when(x == 1)
            def _():
                _fold_remote(b, HH + hh)

    for b in range(B):
        x_rdmas[2 * b].wait_recv()
        x_rdmas[2 * b + 1].wait_recv()
        for hh in range(HH):
            @pl.when(x == 0)
            def _():
                _fold_remote(b, HH + hh)

            @pl.when(x == 1)
            def _():
                _fold_remote(b, hh)

    for i in range(NCHUNK):
        y_rdmas[i].wait_send()
        x_rdmas[i].wait_send()


def kernel(Q, K, V):
    Qt = Q.astype(jnp.bfloat16).transpose(0, 2, 3, 1)
    Kt = K.astype(jnp.bfloat16).transpose(0, 2, 3, 1)
    Vt = V.astype(jnp.bfloat16).transpose(0, 2, 3, 1)
    kvt = jnp.stack([Kt, Vt])

    out = pl.pallas_call(
        _body,
        in_specs=[pl.BlockSpec(memory_space=pltpu.VMEM),
                  pl.BlockSpec(memory_space=pltpu.VMEM)],
        out_specs=pl.BlockSpec(memory_space=pltpu.VMEM),
        out_shape=jax.ShapeDtypeStruct((B, H, D, SQ), jnp.float32),
        scratch_shapes=[
            pltpu.VMEM((2, B, H, D, SQ), jnp.bfloat16),
            pltpu.VMEM((B * H, SQ), jnp.float32),
            pltpu.SemaphoreType.DMA((NCHUNK,)),
            pltpu.SemaphoreType.DMA((NCHUNK,)),
            pltpu.SemaphoreType.DMA((NCHUNK,)),
            pltpu.SemaphoreType.DMA((NCHUNK,)),
        ],
        compiler_params=pltpu.CompilerParams(collective_id=0),
    )(Qt, kvt)
    return out.transpose(0, 3, 1, 2)


# device time: 54715 ns/iter; 2.9885x vs baseline; 1.1256x over previous
import jax
import jax.numpy as jnp
from jax import lax
from jax.experimental import pallas as pl
from jax.experimental.pallas import tpu as pltpu

B, SQ, H, D = 4, 256, 16, 64
HH = H // 2
SCALE = D ** -0.5
NCHUNK = 2 * B


def _part(q, k, v):
    s_t = lax.dot_general(k, q, (((0,), (0,)), ((), ())),
                          preferred_element_type=jnp.float32)
    p = jnp.exp(s_t)
    l = jnp.sum(p, axis=0, keepdims=True)
    o = lax.dot_general(v, p.astype(jnp.bfloat16), (((1,), (0,)), ((), ())),
                        preferred_element_type=jnp.float32)
    return o, l


def _body(q_ref, kv_ref, out_ref, kvr_ref, l_ref,
          ys_sem, yr_sem, xs_sem, xr_sem):
    x = lax.axis_index("x")
    y = lax.axis_index("y")
    z = lax.axis_index("z")
    ynbr = (x, 1 - y, z)
    xnbr = (1 - x, y, z)
    my_h = pl.ds(x * HH, HH)
    ot_h = pl.ds((1 - x) * HH, HH)

    bar = pltpu.get_barrier_semaphore()
    pl.semaphore_signal(bar, inc=1, device_id=ynbr,
                        device_id_type=pl.DeviceIdType.MESH)
    pl.semaphore_signal(bar, inc=1, device_id=xnbr,
                        device_id_type=pl.DeviceIdType.MESH)
    pl.semaphore_wait(bar, 2)

    y_rdmas = []
    for b in range(B):
        for kvi in range(2):
            i = 2 * b + kvi
            r = pltpu.make_async_remote_copy(
                src_ref=kv_ref.at[kvi, b, my_h],
                dst_ref=kvr_ref.at[kvi, b, my_h],
                send_sem=ys_sem.at[i], recv_sem=yr_sem.at[i],
                device_id=ynbr, device_id_type=pl.DeviceIdType.MESH)
            r.start()
            y_rdmas.append(r)

    x_rdmas = []
    for b in range(B):
        for h in range(H):
            q = q_ref[b, h] * SCALE
            o1, l1 = _part(q, kv_ref[0, b, h], kv_ref[1, b, h])
            out_ref[b, h] = o1
            l_ref[pl.ds(b * H + h, 1)] = l1
        for kvi in range(2):
            i = 2 * b + kvi
            y_rdmas[i].wait_recv()
            r = pltpu.make_async_remote_copy(
                src_ref=kvr_ref.at[kvi, b, my_h],
                dst_ref=kvr_ref.at[kvi, b, my_h],
                send_sem=xs_sem.at[i], recv_sem=xr_sem.at[i],
                device_id=xnbr, device_id_type=pl.DeviceIdType.MESH)
            r.start()
            x_rdmas.append(r)

        def _fold_remote(b, h):
            q = q_ref[b, h] * SCALE
            o2, l2 = _part(q, kvr_ref[0, b, h], kvr_ref[1, b, h])
            bh = pl.ds(b * H + h, 1)
            out_ref[b, h] = (out_ref[b, h] + o2) * (1.0 / (l_ref[bh] + l2))

        for hh in range(HH):
            @pl.when(x == 0)
            def _():
                _fold_remote(b, hh)

            @pl.when(x == 1)
            def _():
                _fold_remote(b, HH + hh)

    for b in range(B):
        x_rdmas[2 * b].wait_recv()
        x_rdmas[2 * b + 1].wait_recv()
        for hh in range(HH):
            @pl.when(x == 0)
            def _():
                _fold_remote(b, HH + hh)

            @pl.when(x == 1)
            def _():
                _fold_remote(b, hh)

    for i in range(NCHUNK):
        y_rdmas[i].wait_send()
        x_rdmas[i].wait_send()


def kernel(Q, K, V):
    Qt = Q.astype(jnp.bfloat16).transpose(0, 2, 3, 1)
    Kt = K.astype(jnp.bfloat16).transpose(0, 2, 3, 1)
    Vt = V.astype(jnp.bfloat16).transpose(0, 2, 3, 1)
    kvt = jnp.stack([Kt, Vt])

    out = pl.pallas_call(
        _body,
        in_specs=[pl.BlockSpec(memory_space=pltpu.VMEM),
                  pl.BlockSpec(memory_space=pltpu.VMEM)],
        out_specs=pl.BlockSpec(memory_space=pltpu.VMEM),
        out_shape=jax.ShapeDtypeStruct((B, H, D, SQ), jnp.float32),
        scratch_shapes=[
            pltpu.VMEM((2, B, H, D, SQ), jnp.bfloat16),
            pltpu.VMEM((B * H, SQ), jnp.float32),
            pltpu.SemaphoreType.DMA((NCHUNK,)),
            pltpu.SemaphoreType.DMA((NCHUNK,)),
            pltpu.SemaphoreType.DMA((NCHUNK,)),
            pltpu.SemaphoreType.DMA((NCHUNK,)),
        ],
        compiler_params=pltpu.CompilerParams(collective_id=0),
    )(Qt, kvt)
    return out.transpose(0, 3, 1, 2)


# device time: 43873 ns/iter; 3.7270x vs baseline; 1.2471x over previous
import jax
import jax.numpy as jnp
from jax import lax
from jax.experimental import pallas as pl
from jax.experimental.pallas import tpu as pltpu

B, SQ, H, D = 4, 256, 16, 64
HQ = H // 4
SCALE = D ** -0.5
MESH = pl.DeviceIdType.MESH


def _part(q, k, v):
    s_t = lax.dot_general(k, q, (((0,), (0,)), ((), ())),
                          preferred_element_type=jnp.float32)
    p = jnp.exp(s_t)
    l = jnp.sum(p, axis=0, keepdims=True)
    o = lax.dot_general(v, p.astype(jnp.bfloat16), (((1,), (0,)), ((), ())),
                        preferred_element_type=jnp.float32)
    return o, l


def _body(q_ref, kv_ref, out_ref, kvr_ref, l_ref,
          y_s, y_r, xf_s, xf_r, zf_s, zf_r, xh_s, xh_r, zh_s, zh_r):
    x = lax.axis_index("x")
    y = lax.axis_index("y")
    z = lax.axis_index("z")
    ynbr = (x, 1 - y, z)
    xnbr = (1 - x, y, z)
    znbr = (x, y, 1 - z)
    qb = (2 * x + z) * HQ
    xqb = (2 * (1 - x) + z) * HQ
    zqb = (2 * x + (1 - z)) * HQ
    dqb = (2 * (1 - x) + (1 - z)) * HQ

    bar = pltpu.get_barrier_semaphore()
    for nbr in (ynbr, xnbr, znbr):
        pl.semaphore_signal(bar, inc=1, device_id=nbr, device_id_type=MESH)
    pl.semaphore_wait(bar, 3)

    y_rdmas = []
    for b in range(B):
        r = pltpu.make_async_remote_copy(
            src_ref=kv_ref.at[:, b, pl.ds(qb, HQ)],
            dst_ref=kvr_ref.at[:, b, pl.ds(qb, HQ)],
            send_sem=y_s.at[b], recv_sem=y_r.at[b],
            device_id=ynbr, device_id_type=MESH)
        r.start()
        y_rdmas.append(r)

    def _fold(b, h):
        q = q_ref[b, h] * SCALE
        o2, l2 = _part(q, kvr_ref[0, b, h], kvr_ref[1, b, h])
        bh = pl.ds(b * H + h, 1)
        out_ref[b, h] = (out_ref[b, h] + o2) * (1.0 / (l_ref[bh] + l2))

    xf_rdmas, zf_rdmas = [], []
    for b in range(B):
        for h in range(H):
            q = q_ref[b, h] * SCALE
            o1, l1 = _part(q, kv_ref[0, b, h], kv_ref[1, b, h])
            out_ref[b, h] = o1
            l_ref[pl.ds(b * H + h, 1)] = l1
        y_rdmas[b].wait_recv()
        for lst, sems, nbr in ((xf_rdmas, (xf_s, xf_r), xnbr),
                               (zf_rdmas, (zf_s, zf_r), znbr)):
            r = pltpu.make_async_remote_copy(
                src_ref=kvr_ref.at[:, b, pl.ds(qb, HQ)],
                dst_ref=kvr_ref.at[:, b, pl.ds(qb, HQ)],
                send_sem=sems[0].at[b], recv_sem=sems[1].at[b],
                device_id=nbr, device_id_type=MESH)
            r.start()
            lst.append(r)
        for j in range(HQ):
            _fold(b, qb + j)

    xh_rdmas, zh_rdmas = [], []
    for b in range(B):
        xf_rdmas[b].wait_recv()
        r = pltpu.make_async_remote_copy(
            src_ref=kvr_ref.at[:, b, pl.ds(xqb, HQ // 2)],
            dst_ref=kvr_ref.at[:, b, pl.ds(xqb, HQ // 2)],
            send_sem=zh_s.at[b], recv_sem=zh_r.at[b],
            device_id=znbr, device_id_type=MESH)
        r.start()
        zh_rdmas.append(r)
        for j in range(HQ):
            _fold(b, xqb + j)

        zf_rdmas[b].wait_recv()
        r = pltpu.make_async_remote_copy(
            src_ref=kvr_ref.at[:, b, pl.ds(zqb + HQ // 2, HQ // 2)],
            dst_ref=kvr_ref.at[:, b, pl.ds(zqb + HQ // 2, HQ // 2)],
            send_sem=xh_s.at[b], recv_sem=xh_r.at[b],
            device_id=xnbr, device_id_type=MESH)
        r.start()
        xh_rdmas.append(r)
        for j in range(HQ):
            _fold(b, zqb + j)

    for b in range(B):
        zh_rdmas[b].wait_recv()
        xh_rdmas[b].wait_recv()
        for j in range(HQ):
            _fold(b, dqb + j)

    for b in range(B):
        for r in (y_rdmas[b], xf_rdmas[b], zf_rdmas[b],
                  xh_rdmas[b], zh_rdmas[b]):
            r.wait_send()


def kernel(Q, K, V):
    Qt = Q.astype(jnp.bfloat16).transpose(0, 2, 3, 1)
    Kt = K.astype(jnp.bfloat16).transpose(0, 2, 3, 1)
    Vt = V.astype(jnp.bfloat16).transpose(0, 2, 3, 1)
    kvt = jnp.stack([Kt, Vt])

    out = pl.pallas_call(
        _body,
        in_specs=[pl.BlockSpec(memory_space=pltpu.VMEM),
                  pl.BlockSpec(memory_space=pltpu.VMEM)],
        out_specs=pl.BlockSpec(memory_space=pltpu.VMEM),
        out_shape=jax.ShapeDtypeStruct((B, H, D, SQ), jnp.float32),
        scratch_shapes=[
            pltpu.VMEM((2, B, H, D, SQ), jnp.bfloat16),
            pltpu.VMEM((B * H, SQ), jnp.float32),
        ] + [pltpu.SemaphoreType.DMA((B,)) for _ in range(10)],
        compiler_params=pltpu.CompilerParams(collective_id=0),
    )(Qt, kvt)
    return out.transpose(0, 3, 1, 2)


# device time: 42663 ns/iter; 3.8327x vs baseline; 1.0284x over previous
import jax
import jax.numpy as jnp
from jax import lax
from jax.experimental import pallas as pl
from jax.experimental.pallas import tpu as pltpu

B, SQ, H, D = 4, 256, 16, 64
HQ = H // 4
SCALE = D ** -0.5
MESH = pl.DeviceIdType.MESH


def _body(q_ref, kv_ref, out_ref, kvr_ref,
          y_s, y_r, xf_s, xf_r, zf_s, zf_r, xh_s, xh_r, zh_s, zh_r):
    x = lax.axis_index("x")
    y = lax.axis_index("y")
    z = lax.axis_index("z")
    ynbr = (x, 1 - y, z)
    xnbr = (1 - x, y, z)
    znbr = (x, y, 1 - z)
    qb = (2 * x + z) * HQ
    xqb = (2 * (1 - x) + z) * HQ
    zqb = (2 * x + (1 - z)) * HQ
    dqb = (2 * (1 - x) + (1 - z)) * HQ

    bar = pltpu.get_barrier_semaphore()
    for nbr in (ynbr, xnbr, znbr):
        pl.semaphore_signal(bar, inc=1, device_id=nbr, device_id_type=MESH)
    pl.semaphore_wait(bar, 3)

    y_rdmas = []
    for b in range(B):
        r = pltpu.make_async_remote_copy(
            src_ref=kv_ref.at[:, b, pl.ds(qb, HQ)],
            dst_ref=kvr_ref.at[:, b, pl.ds(qb, HQ)],
            send_sem=y_s.at[b], recv_sem=y_r.at[b],
            device_id=ynbr, device_id_type=MESH)
        r.start()
        y_rdmas.append(r)

    def _fold(b, h):
        q = q_ref[b, h] * SCALE
        k = jnp.concatenate([kv_ref[0, b, h], kvr_ref[0, b, h]], axis=1)
        v = jnp.concatenate([kv_ref[1, b, h], kvr_ref[1, b, h]], axis=1)
        s_t = lax.dot_general(k, q, (((0,), (0,)), ((), ())),
                              preferred_element_type=jnp.float32)
        p = jnp.exp(s_t)
        l = jnp.sum(p, axis=0, keepdims=True)
        o = lax.dot_general(v, p.astype(jnp.bfloat16),
                            (((1,), (0,)), ((), ())),
                            preferred_element_type=jnp.float32)
        out_ref[b, h] = o * (1.0 / l)

    xf_rdmas, zf_rdmas = [], []
    for b in range(B):
        y_rdmas[b].wait_recv()
        for lst, sems, nbr in ((xf_rdmas, (xf_s, xf_r), xnbr),
                               (zf_rdmas, (zf_s, zf_r), znbr)):
            r = pltpu.make_async_remote_copy(
                src_ref=kvr_ref.at[:, b, pl.ds(qb, HQ)],
                dst_ref=kvr_ref.at[:, b, pl.ds(qb, HQ)],
                send_sem=sems[0].at[b], recv_sem=sems[1].at[b],
                device_id=nbr, device_id_type=MESH)
            r.start()
            lst.append(r)
        for j in range(HQ):
            _fold(b, qb + j)

    xh_rdmas, zh_rdmas = [], []
    for b in range(B):
        xf_rdmas[b].wait_recv()
        r = pltpu.make_async_remote_copy(
            src_ref=kvr_ref.at[:, b, pl.ds(xqb, HQ // 2)],
            dst_ref=kvr_ref.at[:, b, pl.ds(xqb, HQ // 2)],
            send_sem=zh_s.at[b], recv_sem=zh_r.at[b],
            device_id=znbr, device_id_type=MESH)
        r.start()
        zh_rdmas.append(r)
        for j in range(HQ):
            _fold(b, xqb + j)

        zf_rdmas[b].wait_recv()
        r = pltpu.make_async_remote_copy(
            src_ref=kvr_ref.at[:, b, pl.ds(zqb + HQ // 2, HQ // 2)],
            dst_ref=kvr_ref.at[:, b, pl.ds(zqb + HQ // 2, HQ // 2)],
            send_sem=xh_s.at[b], recv_sem=xh_r.at[b],
            device_id=xnbr, device_id_type=MESH)
        r.start()
        xh_rdmas.append(r)
        for j in range(HQ):
            _fold(b, zqb + j)

    for b in range(B):
        zh_rdmas[b].wait_recv()
        xh_rdmas[b].wait_recv()
        for j in range(HQ):
            _fold(b, dqb + j)

    for b in range(B):
        for r in (y_rdmas[b], xf_rdmas[b], zf_rdmas[b],
                  xh_rdmas[b], zh_rdmas[b]):
            r.wait_send()


def kernel(Q, K, V):
    Qt = Q.astype(jnp.bfloat16).transpose(0, 2, 3, 1)
    Kt = K.astype(jnp.bfloat16).transpose(0, 2, 3, 1)
    Vt = V.astype(jnp.bfloat16).transpose(0, 2, 3, 1)
    kvt = jnp.stack([Kt, Vt])

    out = pl.pallas_call(
        _body,
        in_specs=[pl.BlockSpec(memory_space=pltpu.VMEM),
                  pl.BlockSpec(memory_space=pltpu.VMEM)],
        out_specs=pl.BlockSpec(memory_space=pltpu.VMEM),
        out_shape=jax.ShapeDtypeStruct((B, H, D, SQ), jnp.float32),
        scratch_shapes=[
            pltpu.VMEM((2, B, H, D, SQ), jnp.bfloat16),
        ] + [pltpu.SemaphoreType.DMA((B,)) for _ in range(10)],
        compiler_params=pltpu.CompilerParams(collective_id=0),
    )(Qt, kvt)
    return out.transpose(0, 3, 1, 2)


# device time: 41909 ns/iter; 3.9017x vs baseline; 1.0180x over previous
import jax
import jax.numpy as jnp
from jax import lax
from jax.experimental import pallas as pl
from jax.experimental.pallas import tpu as pltpu

B, SQ, H, D = 4, 256, 16, 64
HQ = H // 4
SCALE = D ** -0.5
MESH = pl.DeviceIdType.MESH


def _body(q_ref, kv_ref, out_ref, kvr_ref,
          y_s, y_r, xf_s, xf_r, zf_s, zf_r, xh_s, xh_r, zh_s, zh_r):
    x = lax.axis_index("x")
    y = lax.axis_index("y")
    z = lax.axis_index("z")
    ynbr = (x, 1 - y, z)
    xnbr = (1 - x, y, z)
    znbr = (x, y, 1 - z)
    qb = (2 * x + z) * HQ
    xqb = (2 * (1 - x) + z) * HQ
    zqb = (2 * x + (1 - z)) * HQ
    dqb = (2 * (1 - x) + (1 - z)) * HQ

    bar = pltpu.get_barrier_semaphore()
    for nbr in (ynbr, xnbr, znbr):
        pl.semaphore_signal(bar, inc=1, device_id=nbr, device_id_type=MESH)
    pl.semaphore_wait(bar, 3)

    y_rdmas = []
    for b in range(B):
        r = pltpu.make_async_remote_copy(
            src_ref=kv_ref.at[:, b, pl.ds(qb, HQ)],
            dst_ref=kvr_ref.at[:, b, pl.ds(qb, HQ)],
            send_sem=y_s.at[b], recv_sem=y_r.at[b],
            device_id=ynbr, device_id_type=MESH)
        r.start()
        y_rdmas.append(r)

    def _fold(b, h):
        q = q_ref[b, h] * SCALE
        k = jnp.concatenate([kv_ref[0, b, h], kvr_ref[0, b, h]], axis=1)
        v = jnp.concatenate([kv_ref[1, b, h], kvr_ref[1, b, h]], axis=1)
        s_t = lax.dot_general(k, q, (((0,), (0,)), ((), ())),
                              preferred_element_type=jnp.float32)
        p = jnp.exp(s_t)
        l = jnp.sum(p, axis=0, keepdims=True)
        o = lax.dot_general(v, p.astype(jnp.bfloat16),
                            (((1,), (0,)), ((), ())),
                            preferred_element_type=jnp.float32)
        out_ref[b, h] = o * (1.0 / l)

    xf_rdmas, zf_rdmas = [], []
    for b in range(B):
        y_rdmas[b].wait_recv()
        for lst, sems, nbr in ((xf_rdmas, (xf_s, xf_r), xnbr),
                               (zf_rdmas, (zf_s, zf_r), znbr)):
            r = pltpu.make_async_remote_copy(
                src_ref=kvr_ref.at[:, b, pl.ds(qb, HQ)],
                dst_ref=kvr_ref.at[:, b, pl.ds(qb, HQ)],
                send_sem=sems[0].at[b], recv_sem=sems[1].at[b],
                device_id=nbr, device_id_type=MESH)
            r.start()
            lst.append(r)
        for j in range(HQ):
            _fold(b, qb + j)

    xh_rdmas, zh_rdmas = [], []
    for b in range(B):
        xf_rdmas[b].wait_recv()
        r = pltpu.make_async_remote_copy(
            src_ref=kvr_ref.at[:, b, pl.ds(xqb, HQ // 2)],
            dst_ref=kvr_ref.at[:, b, pl.ds(xqb, HQ // 2)],
            send_sem=zh_s.at[b], recv_sem=zh_r.at[b],
            device_id=znbr, device_id_type=MESH)
        r.start()
        zh_rdmas.append(r)

        zf_rdmas[b].wait_recv()
        r = pltpu.make_async_remote_copy(
            src_ref=kvr_ref.at[:, b, pl.ds(zqb + HQ // 2, HQ // 2)],
            dst_ref=kvr_ref.at[:, b, pl.ds(zqb + HQ // 2, HQ // 2)],
            send_sem=xh_s.at[b], recv_sem=xh_r.at[b],
            device_id=xnbr, device_id_type=MESH)
        r.start()
        xh_rdmas.append(r)
        for j in range(HQ):
            _fold(b, xqb + j)
        for j in range(HQ):
            _fold(b, zqb + j)

    for b in range(B):
        zh_rdmas[b].wait_recv()
        xh_rdmas[b].wait_recv()
        for j in range(HQ):
            _fold(b, dqb + j)

    for b in range(B):
        for r in (y_rdmas[b], xf_rdmas[b], zf_rdmas[b],
                  xh_rdmas[b], zh_rdmas[b]):
            r.wait_send()


def kernel(Q, K, V):
    Qt = Q.astype(jnp.bfloat16).transpose(0, 2, 3, 1)
    Kt = K.astype(jnp.bfloat16).transpose(0, 2, 3, 1)
    Vt = V.astype(jnp.bfloat16).transpose(0, 2, 3, 1)
    kvt = jnp.stack([Kt, Vt])

    out = pl.pallas_call(
        _body,
        in_specs=[pl.BlockSpec(memory_space=pltpu.VMEM),
                  pl.BlockSpec(memory_space=pltpu.VMEM)],
        out_specs=pl.BlockSpec(memory_space=pltpu.VMEM),
        out_shape=jax.ShapeDtypeStruct((B, H, D, SQ), jnp.float32),
        scratch_shapes=[
            pltpu.VMEM((2, B, H, D, SQ), jnp.bfloat16),
        ] + [pltpu.SemaphoreType.DMA((B,)) for _ in range(10)],
        compiler_params=pltpu.CompilerParams(collective_id=0),
    )(Qt, kvt)
    return out.transpose(0, 3, 1, 2)
